# Initial kernel scaffold; baseline (speedup 1.0000x reference)
import functools

import jax
import jax.numpy as jnp
from jax import lax
from jax.experimental import pallas as pl
from jax.experimental.pallas import tpu as pltpu
from jax.experimental.pallas import tpu_sc as plsc

N = 10000
E = 320000
D = 128
H = 128
G = 64

NC = 2
NS = 16
NW = NC * NS

K = 128
CHUNKS = 79
EPW = CHUNKS * K
EP = NW * EPW
NPAD = N + 16
RPS = N // NS

_mesh = plsc.VectorSubcoreMesh(core_axis_name="c", subcore_axis_name="s")


def _agg_body(x_hbm, src_hbm, dst_hbm, out_hbm, src_v, dst_v, buf_a, buf_b,
              acc_sh, sem_a, sem_b):
    cid = lax.axis_index("c")
    sid = lax.axis_index("s")
    wid = cid * NS + sid

    pltpu.sync_copy(src_hbm.at[wid], src_v)
    pltpu.sync_copy(dst_hbm.at[wid], dst_v)

    @pl.loop(0, K)
    def _(r):
        for l in range(D // 16):
            buf_a[r, pl.ds(l * 16, 16)] = jnp.zeros((16,), jnp.float32)

    for j in range(5):
        pltpu.sync_copy(buf_a.at[pl.ds(0, RPS // 5)],
                        acc_sh.at[pl.ds(sid * RPS + j * (RPS // 5), RPS // 5)])
    plsc.subcore_barrier()

    pltpu.async_copy(x_hbm.at[src_v.at[0]], buf_a, sem_a)

    @pl.loop(0, CHUNKS - 2, step=2)
    def _(c):
        pltpu.make_async_copy(x_hbm.at[src_v.at[c]], buf_a, sem_a).wait()
        pltpu.async_copy(x_hbm.at[src_v.at[c + 1]], buf_b, sem_b)
        pltpu.sync_copy(buf_a, acc_sh.at[dst_v.at[c]], add=True)
        pltpu.make_async_copy(x_hbm.at[src_v.at[c + 1]], buf_b, sem_b).wait()
        pltpu.async_copy(x_hbm.at[src_v.at[c + 2]], buf_a, sem_a)
        pltpu.sync_copy(buf_b, acc_sh.at[dst_v.at[c + 1]], add=True)

    pltpu.make_async_copy(x_hbm.at[src_v.at[CHUNKS - 1]], buf_a, sem_a).wait()
    pltpu.sync_copy(buf_a, acc_sh.at[dst_v.at[CHUNKS - 1]], add=True)

    plsc.subcore_barrier()
    pltpu.sync_copy(acc_sh.at[pl.ds(sid * RPS, RPS)],
                    out_hbm.at[cid, pl.ds(sid * RPS, RPS)])


_agg = functools.partial(
    pl.kernel,
    out_type=jax.ShapeDtypeStruct((NC, N, D), jnp.float32),
    mesh=_mesh,
    scratch_types=[
        pltpu.VMEM((CHUNKS, K), jnp.int32),
        pltpu.VMEM((CHUNKS, K), jnp.int32),
        pltpu.VMEM((K, D), jnp.float32),
        pltpu.VMEM((K, D), jnp.float32),
        pltpu.VMEM_SHARED((NPAD, D), jnp.float32),
        pltpu.SemaphoreType.DMA,
        pltpu.SemaphoreType.DMA,
    ],
)(_agg_body)


def _mlp_body(p_ref, h_ref, w1_ref, b1_ref, w2_ref, b2_ref, o_ref):
    z = p_ref[0] + p_ref[1] + h_ref[...]
    y = jnp.maximum(
        jnp.dot(z, w1_ref[...], preferred_element_type=jnp.float32) + b1_ref[...], 0.0)
    o_ref[...] = jnp.maximum(
        jnp.dot(y, w2_ref[...], preferred_element_type=jnp.float32) + b2_ref[...], 0.0)


def _mlp_pool_body(p_ref, h_ref, w1_ref, b1_ref, w2_ref, b2_ref, bat_ref,
                   o_ref, pool_ref, acc_ref, cnt_ref):
    i = pl.program_id(0)
    z = p_ref[0] + p_ref[1] + h_ref[...]
    y = jnp.maximum(
        jnp.dot(z, w1_ref[...], preferred_element_type=jnp.float32) + b1_ref[...], 0.0)
    h2 = jnp.maximum(
        jnp.dot(y, w2_ref[...], preferred_element_type=jnp.float32) + b2_ref[...], 0.0)
    o_ref[...] = h2

    bn = h2.shape[0]
    ids = bat_ref[0]
    gids = lax.broadcasted_iota(jnp.int32, (G, bn), 0)
    oh = (gids == jnp.broadcast_to(ids, (G, bn))).astype(jnp.float32)
    part = jnp.dot(oh, h2, preferred_element_type=jnp.float32)
    csum = jnp.sum(oh, axis=1, keepdims=True)

    @pl.when(i == 0)
    def _():
        acc_ref[...] = part
        cnt_ref[...] = csum

    @pl.when(i > 0)
    def _():
        acc_ref[...] += part
        cnt_ref[...] += csum

    pool_ref[...] = acc_ref[...] / jnp.maximum(cnt_ref[...], 1.0)


_BN = 1000


def _mlp(p, h, w1t, b1, w2t, b2):
    return pl.pallas_call(
        _mlp_body,
        grid=(N // _BN,),
        in_specs=[
            pl.BlockSpec((NC, _BN, D), lambda i: (0, i, 0)),
            pl.BlockSpec((_BN, D), lambda i: (i, 0)),
            pl.BlockSpec((D, H), lambda i: (0, 0)),
            pl.BlockSpec((1, H), lambda i: (0, 0)),
            pl.BlockSpec((H, H), lambda i: (0, 0)),
            pl.BlockSpec((1, H), lambda i: (0, 0)),
        ],
        out_specs=pl.BlockSpec((_BN, D), lambda i: (i, 0)),
        out_shape=jax.ShapeDtypeStruct((N, D), jnp.float32),
    )(p, h, w1t, b1, w2t, b2)


def _mlp_pool(p, h, w1t, b1, w2t, b2, bat3):
    return pl.pallas_call(
        _mlp_pool_body,
        grid=(N // _BN,),
        in_specs=[
            pl.BlockSpec((NC, _BN, D), lambda i: (0, i, 0)),
            pl.BlockSpec((_BN, D), lambda i: (i, 0)),
            pl.BlockSpec((D, H), lambda i: (0, 0)),
            pl.BlockSpec((1, H), lambda i: (0, 0)),
            pl.BlockSpec((H, H), lambda i: (0, 0)),
            pl.BlockSpec((1, H), lambda i: (0, 0)),
            pl.BlockSpec((1, 1, _BN), lambda i: (i, 0, 0)),
        ],
        out_specs=[
            pl.BlockSpec((_BN, D), lambda i: (i, 0)),
            pl.BlockSpec((G, D), lambda i: (0, 0)),
        ],
        out_shape=[
            jax.ShapeDtypeStruct((N, D), jnp.float32),
            jax.ShapeDtypeStruct((G, D), jnp.float32),
        ],
        scratch_shapes=[
            pltpu.VMEM((G, D), jnp.float32),
            pltpu.VMEM((G, 1), jnp.float32),
        ],
    )(p, h, w1t, b1, w2t, b2, bat3)


def kernel(x, edge_index, batch, W1, b1, W2, b2, W3, b3, W4, b4):
    src = edge_index[0]
    dst = edge_index[1]
    pad = EP - E
    src3 = jnp.concatenate([src, jnp.zeros((pad,), jnp.int32)]).reshape(NW, CHUNKS, K)
    dst3 = jnp.concatenate([dst, jnp.full((pad,), N, jnp.int32)]).reshape(NW, CHUNKS, K)
    bat3 = batch.reshape(N // _BN, 1, _BN)

    w1t, w2t, w3t, w4t = W1.T, W2.T, W3.T, W4.T
    b1r, b2r, b3r, b4r = (b.reshape(1, H) for b in (b1, b2, b3, b4))

    p1 = _agg(x, src3, dst3)
    h1 = _mlp(p1, x, w1t, b1r, w2t, b2r)
    p2 = _agg(h1, src3, dst3)
    h2, pool = _mlp_pool(p2, h1, w3t, b3r, w4t, b4r, bat3)
    return (pool, h2)


# speedup vs baseline: 2.8722x; 2.8722x over previous
import functools

import jax
import jax.numpy as jnp
from jax import lax
from jax.experimental import pallas as pl
from jax.experimental.pallas import tpu as pltpu
from jax.experimental.pallas import tpu_sc as plsc

N = 10000
E = 320000
D = 128
H = 128
G = 64

NC = 2
NS = 16
NW = NC * NS

K = 128
CHUNKS = 80
IB = 16
NG = CHUNKS // IB
EPW = CHUNKS * K
EP = NW * EPW
NPAD = 10240
RPS = NPAD // NS

_mesh = plsc.VectorSubcoreMesh(core_axis_name="c", subcore_axis_name="s")


def _agg_body(x_hbm, src_hbm, dst_hbm, out_hbm, src_v, dst_v, buf_a, buf_b,
              acc_sh, sem_a, sem_b):
    cid = lax.axis_index("c")
    sid = lax.axis_index("s")
    wid = cid * NS + sid

    @pl.loop(0, K)
    def _(r):
        for l in range(D // 16):
            buf_a[r, pl.ds(l * 16, 16)] = jnp.zeros((16,), jnp.float32)

    for j in range(RPS // K):
        pltpu.sync_copy(buf_a,
                        acc_sh.at[pl.ds(sid * RPS + j * K, K)])
    plsc.subcore_barrier()

    @pl.loop(0, NG)
    def _(g):
        pltpu.sync_copy(src_hbm.at[wid, pl.ds(g * IB, IB)], src_v)
        pltpu.sync_copy(dst_hbm.at[wid, pl.ds(g * IB, IB)], dst_v)
        pltpu.async_copy(x_hbm.at[src_v.at[0]], buf_a, sem_a)

        @pl.loop(0, IB - 2, step=2)
        def _(c):
            pltpu.make_async_copy(x_hbm.at[src_v.at[c]], buf_a, sem_a).wait()
            pltpu.async_copy(x_hbm.at[src_v.at[c + 1]], buf_b, sem_b)
            pltpu.sync_copy(buf_a, acc_sh.at[dst_v.at[c]], add=True)
            pltpu.make_async_copy(x_hbm.at[src_v.at[c + 1]], buf_b, sem_b).wait()
            pltpu.async_copy(x_hbm.at[src_v.at[c + 2]], buf_a, sem_a)
            pltpu.sync_copy(buf_b, acc_sh.at[dst_v.at[c + 1]], add=True)

        pltpu.make_async_copy(x_hbm.at[src_v.at[IB - 2]], buf_a, sem_a).wait()
        pltpu.async_copy(x_hbm.at[src_v.at[IB - 1]], buf_b, sem_b)
        pltpu.sync_copy(buf_a, acc_sh.at[dst_v.at[IB - 2]], add=True)
        pltpu.make_async_copy(x_hbm.at[src_v.at[IB - 1]], buf_b, sem_b).wait()
        pltpu.sync_copy(buf_b, acc_sh.at[dst_v.at[IB - 1]], add=True)

    plsc.subcore_barrier()
    pltpu.sync_copy(acc_sh.at[pl.ds(sid * RPS, RPS)],
                    out_hbm.at[cid, pl.ds(sid * RPS, RPS)])


_agg = functools.partial(
    pl.kernel,
    out_type=jax.ShapeDtypeStruct((NC, NPAD, D), jnp.float32),
    mesh=_mesh,
    scratch_types=[
        pltpu.VMEM((IB, K), jnp.int32),
        pltpu.VMEM((IB, K), jnp.int32),
        pltpu.VMEM((K, D), jnp.float32),
        pltpu.VMEM((K, D), jnp.float32),
        pltpu.VMEM_SHARED((NPAD, D), jnp.float32),
        pltpu.SemaphoreType.DMA,
        pltpu.SemaphoreType.DMA,
    ],
)(_agg_body)


def _mlp_body(p_ref, h_ref, w1_ref, b1_ref, w2_ref, b2_ref, o_ref):
    z = p_ref[0] + p_ref[1] + h_ref[...]
    y = jnp.maximum(
        jnp.dot(z, w1_ref[...], preferred_element_type=jnp.float32) + b1_ref[...], 0.0)
    o_ref[...] = jnp.maximum(
        jnp.dot(y, w2_ref[...], preferred_element_type=jnp.float32) + b2_ref[...], 0.0)


def _mlp_pool_body(p_ref, h_ref, w1_ref, b1_ref, w2_ref, b2_ref, bat_ref,
                   o_ref, pool_ref, acc_ref, cnt_ref):
    i = pl.program_id(0)
    z = p_ref[0] + p_ref[1] + h_ref[...]
    y = jnp.maximum(
        jnp.dot(z, w1_ref[...], preferred_element_type=jnp.float32) + b1_ref[...], 0.0)
    h2 = jnp.maximum(
        jnp.dot(y, w2_ref[...], preferred_element_type=jnp.float32) + b2_ref[...], 0.0)
    o_ref[...] = h2

    bn = h2.shape[0]
    ids = bat_ref[0]
    gids = lax.broadcasted_iota(jnp.int32, (G, bn), 0)
    oh = (gids == jnp.broadcast_to(ids, (G, bn))).astype(jnp.float32)
    part = jnp.dot(oh, h2, preferred_element_type=jnp.float32)
    csum = jnp.sum(oh, axis=1, keepdims=True)

    @pl.when(i == 0)
    def _():
        acc_ref[...] = part
        cnt_ref[...] = csum

    @pl.when(i > 0)
    def _():
        acc_ref[...] += part
        cnt_ref[...] += csum

    pool_ref[...] = acc_ref[...] / jnp.maximum(cnt_ref[...], 1.0)


_BN = 1000


def _mlp(p, h, w1t, b1, w2t, b2):
    return pl.pallas_call(
        _mlp_body,
        grid=(N // _BN,),
        in_specs=[
            pl.BlockSpec((NC, _BN, D), lambda i: (0, i, 0)),
            pl.BlockSpec((_BN, D), lambda i: (i, 0)),
            pl.BlockSpec((D, H), lambda i: (0, 0)),
            pl.BlockSpec((1, H), lambda i: (0, 0)),
            pl.BlockSpec((H, H), lambda i: (0, 0)),
            pl.BlockSpec((1, H), lambda i: (0, 0)),
        ],
        out_specs=pl.BlockSpec((_BN, D), lambda i: (i, 0)),
        out_shape=jax.ShapeDtypeStruct((N, D), jnp.float32),
    )(p, h, w1t, b1, w2t, b2)


def _mlp_pool(p, h, w1t, b1, w2t, b2, bat3):
    return pl.pallas_call(
        _mlp_pool_body,
        grid=(N // _BN,),
        in_specs=[
            pl.BlockSpec((NC, _BN, D), lambda i: (0, i, 0)),
            pl.BlockSpec((_BN, D), lambda i: (i, 0)),
            pl.BlockSpec((D, H), lambda i: (0, 0)),
            pl.BlockSpec((1, H), lambda i: (0, 0)),
            pl.BlockSpec((H, H), lambda i: (0, 0)),
            pl.BlockSpec((1, H), lambda i: (0, 0)),
            pl.BlockSpec((1, 1, _BN), lambda i: (i, 0, 0)),
        ],
        out_specs=[
            pl.BlockSpec((_BN, D), lambda i: (i, 0)),
            pl.BlockSpec((G, D), lambda i: (0, 0)),
        ],
        out_shape=[
            jax.ShapeDtypeStruct((N, D), jnp.float32),
            jax.ShapeDtypeStruct((G, D), jnp.float32),
        ],
        scratch_shapes=[
            pltpu.VMEM((G, D), jnp.float32),
            pltpu.VMEM((G, 1), jnp.float32),
        ],
    )(p, h, w1t, b1, w2t, b2, bat3)


def kernel(x, edge_index, batch, W1, b1, W2, b2, W3, b3, W4, b4):
    src = edge_index[0]
    dst = edge_index[1]
    pad = EP - E
    src3 = jnp.concatenate([src, jnp.zeros((pad,), jnp.int32)]).reshape(NW, CHUNKS, K)
    dst3 = jnp.concatenate([dst, jnp.full((pad,), N, jnp.int32)]).reshape(NW, CHUNKS, K)
    bat3 = batch.reshape(N // _BN, 1, _BN)

    w1t, w2t, w3t, w4t = W1.T, W2.T, W3.T, W4.T
    b1r, b2r, b3r, b4r = (b.reshape(1, H) for b in (b1, b2, b3, b4))

    p1 = _agg(x, src3, dst3)
    h1 = _mlp(p1, x, w1t, b1r, w2t, b2r)
    p2 = _agg(h1, src3, dst3)
    h2, pool = _mlp_pool(p2, h1, w3t, b3r, w4t, b4r, bat3)
    return (pool, h2)


# speedup vs baseline: 3.6090x; 1.2565x over previous
import functools

import jax
import jax.numpy as jnp
from jax import lax
from jax.experimental import pallas as pl
from jax.experimental.pallas import tpu as pltpu
from jax.experimental.pallas import tpu_sc as plsc

N = 10000
E = 320000
D = 128
H = 128
G = 64

NC = 2
NS = 16
NW = NC * NS

K = 128
CHUNKS = 80
IB = 16
NG = CHUNKS // IB
EPW = CHUNKS * K
EP = NW * EPW
NPAD = 10240
RPS = NPAD // NS

_mesh = plsc.VectorSubcoreMesh(core_axis_name="c", subcore_axis_name="s")


def _agg_body(x_hbm, src_hbm, dst_hbm, out_hbm, src_v, dst_v, buf_a, buf_b,
              acc_sh, sem_a, sem_b):
    cid = lax.axis_index("c")
    sid = lax.axis_index("s")
    wid = cid * NS + sid

    @pl.loop(0, K)
    def _(r):
        for l in range(D // 16):
            buf_a[r, pl.ds(l * 16, 16)] = jnp.zeros((16,), jnp.float32)

    for j in range(RPS // K):
        pltpu.sync_copy(buf_a,
                        acc_sh.at[pl.ds(sid * RPS + j * K, K)])
    plsc.subcore_barrier()

    @pl.loop(0, NG)
    def _(g):
        pltpu.sync_copy(src_hbm.at[wid, pl.ds(g * IB, IB)], src_v)
        pltpu.sync_copy(dst_hbm.at[wid, pl.ds(g * IB, IB)], dst_v)
        pltpu.async_copy(x_hbm.at[src_v.at[0]], buf_a, sem_a)

        @pl.loop(0, IB - 2, step=2)
        def _(c):
            pltpu.make_async_copy(x_hbm.at[src_v.at[c]], buf_a, sem_a).wait()
            pltpu.async_copy(x_hbm.at[src_v.at[c + 1]], buf_b, sem_b)
            pltpu.sync_copy(buf_a, acc_sh.at[dst_v.at[c]], add=True)
            pltpu.make_async_copy(x_hbm.at[src_v.at[c + 1]], buf_b, sem_b).wait()
            pltpu.async_copy(x_hbm.at[src_v.at[c + 2]], buf_a, sem_a)
            pltpu.sync_copy(buf_b, acc_sh.at[dst_v.at[c + 1]], add=True)

        pltpu.make_async_copy(x_hbm.at[src_v.at[IB - 2]], buf_a, sem_a).wait()
        pltpu.async_copy(x_hbm.at[src_v.at[IB - 1]], buf_b, sem_b)
        pltpu.sync_copy(buf_a, acc_sh.at[dst_v.at[IB - 2]], add=True)
        pltpu.make_async_copy(x_hbm.at[src_v.at[IB - 1]], buf_b, sem_b).wait()
        pltpu.sync_copy(buf_b, acc_sh.at[dst_v.at[IB - 1]], add=True)

    plsc.subcore_barrier()
    pltpu.sync_copy(acc_sh.at[pl.ds(sid * RPS, RPS)],
                    out_hbm.at[cid, pl.ds(sid * RPS, RPS)])


_agg = functools.partial(
    pl.kernel,
    out_type=jax.ShapeDtypeStruct((NC, NPAD, D), jnp.float32),
    mesh=_mesh,
    scratch_types=[
        pltpu.VMEM((IB, K), jnp.int32),
        pltpu.VMEM((IB, K), jnp.int32),
        pltpu.VMEM((K, D), jnp.float32),
        pltpu.VMEM((K, D), jnp.float32),
        pltpu.VMEM_SHARED((NPAD, D), jnp.float32),
        pltpu.SemaphoreType.DMA,
        pltpu.SemaphoreType.DMA,
    ],
)(_agg_body)


def _mlp_body(p_ref, h_ref, w1_ref, b1_ref, w2_ref, b2_ref, o_ref):
    z = p_ref[0] + p_ref[1] + h_ref[...]
    y = jnp.maximum(
        jnp.dot(z, w1_ref[...], preferred_element_type=jnp.float32) + b1_ref[...], 0.0)
    o_ref[...] = jnp.maximum(
        jnp.dot(y, w2_ref[...], preferred_element_type=jnp.float32) + b2_ref[...], 0.0)


def _mlp_pool_body(p_ref, h_ref, w1_ref, b1_ref, w2_ref, b2_ref, bat_ref,
                   o_ref, pool_ref, acc_ref, cnt_ref):
    i = pl.program_id(0)
    z = p_ref[0] + p_ref[1] + h_ref[...]
    y = jnp.maximum(
        jnp.dot(z, w1_ref[...], preferred_element_type=jnp.float32) + b1_ref[...], 0.0)
    h2 = jnp.maximum(
        jnp.dot(y, w2_ref[...], preferred_element_type=jnp.float32) + b2_ref[...], 0.0)
    o_ref[...] = h2

    bn = h2.shape[0]
    ids = bat_ref[0]
    gids = lax.broadcasted_iota(jnp.int32, (G, bn), 0)
    oh = (gids == jnp.broadcast_to(ids, (G, bn))).astype(jnp.float32)
    part = jnp.dot(oh, h2, preferred_element_type=jnp.float32)
    csum = jnp.sum(oh, axis=1, keepdims=True)

    @pl.when(i == 0)
    def _():
        acc_ref[...] = part
        cnt_ref[...] = csum

    @pl.when(i > 0)
    def _():
        acc_ref[...] += part
        cnt_ref[...] += csum

    pool_ref[...] = acc_ref[...] / jnp.maximum(cnt_ref[...], 1.0)


_BN = 1000


def _mlp(p, h, w1t, b1, w2t, b2):
    return pl.pallas_call(
        _mlp_body,
        grid=(N // _BN,),
        in_specs=[
            pl.BlockSpec((NC, _BN, D), lambda i: (0, i, 0)),
            pl.BlockSpec((_BN, D), lambda i: (i, 0)),
            pl.BlockSpec((D, H), lambda i: (0, 0)),
            pl.BlockSpec((1, H), lambda i: (0, 0)),
            pl.BlockSpec((H, H), lambda i: (0, 0)),
            pl.BlockSpec((1, H), lambda i: (0, 0)),
        ],
        out_specs=pl.BlockSpec((_BN, D), lambda i: (i, 0)),
        out_shape=jax.ShapeDtypeStruct((N, D), jnp.float32),
    )(p, h, w1t, b1, w2t, b2)


def _mlp_pool(p, h, w1t, b1, w2t, b2, bat3):
    return pl.pallas_call(
        _mlp_pool_body,
        grid=(N // _BN,),
        in_specs=[
            pl.BlockSpec((NC, _BN, D), lambda i: (0, i, 0)),
            pl.BlockSpec((_BN, D), lambda i: (i, 0)),
            pl.BlockSpec((D, H), lambda i: (0, 0)),
            pl.BlockSpec((1, H), lambda i: (0, 0)),
            pl.BlockSpec((H, H), lambda i: (0, 0)),
            pl.BlockSpec((1, H), lambda i: (0, 0)),
            pl.BlockSpec((1, 1, _BN), lambda i: (i, 0, 0)),
        ],
        out_specs=[
            pl.BlockSpec((_BN, D), lambda i: (i, 0)),
            pl.BlockSpec((G, D), lambda i: (0, 0)),
        ],
        out_shape=[
            jax.ShapeDtypeStruct((N, D), jnp.float32),
            jax.ShapeDtypeStruct((G, D), jnp.float32),
        ],
        scratch_shapes=[
            pltpu.VMEM((G, D), jnp.float32),
            pltpu.VMEM((G, 1), jnp.float32),
        ],
    )(p, h, w1t, b1, w2t, b2, bat3)


def kernel(x, edge_index, batch, W1, b1, W2, b2, W3, b3, W4, b4):
    src = edge_index[0]
    dst = edge_index[1]
    padw = EPW - E // NW
    src2 = src.reshape(NW, E // NW)
    dst2 = dst.reshape(NW, E // NW)
    pad_src = jnp.zeros((NW, padw), jnp.int32)
    pad_dst = jnp.broadcast_to(
        N + (jnp.arange(padw, dtype=jnp.int32) % (NPAD - N)), (NW, padw))
    src3 = jnp.concatenate([src2, pad_src], axis=1).reshape(NW, CHUNKS, K)
    dst3 = jnp.concatenate([dst2, pad_dst], axis=1).reshape(NW, CHUNKS, K)
    bat3 = batch.reshape(N // _BN, 1, _BN)

    w1t, w2t, w3t, w4t = W1.T, W2.T, W3.T, W4.T
    b1r, b2r, b3r, b4r = (b.reshape(1, H) for b in (b1, b2, b3, b4))

    p1 = _agg(x, src3, dst3)
    h1 = _mlp(p1, x, w1t, b1r, w2t, b2r)
    p2 = _agg(h1, src3, dst3)
    h2, pool = _mlp_pool(p2, h1, w3t, b3r, w4t, b4r, bat3)
    return (pool, h2)


# speedup vs baseline: 3.6495x; 1.0112x over previous
import functools

import jax
import jax.numpy as jnp
from jax import lax
from jax.experimental import pallas as pl
from jax.experimental.pallas import tpu as pltpu
from jax.experimental.pallas import tpu_sc as plsc

N = 10000
E = 320000
D = 128
H = 128
G = 64

NC = 2
NS = 16
NW = NC * NS

K = 64
CHUNKS = 160
IB = 16
NG = CHUNKS // IB
NB = 4
EPW = CHUNKS * K
EP = NW * EPW
NPAD = 10240
RPS = NPAD // NS

_mesh = plsc.VectorSubcoreMesh(core_axis_name="c", subcore_axis_name="s")


def _agg_body(x_hbm, idx_hbm, out_hbm, ix0, ix1, b0, b1, b2, b3, zb, acc_sh,
              sg0, sg1, sg2, sg3, ss0, ss1, ss2, ss3, si, sz):
    cid = lax.axis_index("c")
    sid = lax.axis_index("s")
    wid = cid * NS + sid

    bufs = (b0, b1, b2, b3)
    gsem = (sg0, sg1, sg2, sg3)
    ssem = (ss0, ss1, ss2, ss3)

    def issue_gather(b, ix, row):
        pltpu.async_copy(x_hbm.at[ix.at[row]], bufs[b], gsem[b])

    def wait_gather(b, ix, row):
        pltpu.make_async_copy(x_hbm.at[ix.at[row]], bufs[b], gsem[b]).wait()

    def issue_scat(b, ix, row):
        pltpu.async_copy(bufs[b], acc_sh.at[ix.at[IB + row]], ssem[b], add=True)

    def wait_scat(b, ix, row):
        pltpu.make_async_copy(bufs[b], acc_sh.at[ix.at[IB + row]], ssem[b]).wait()

    def emit_group(ix_cur, ix_nxt, g, first=False, last=False):
        for j in range(IB):
            b = j % NB
            wait_gather(b, ix_cur, j)
            issue_scat(b, ix_cur, j)
            b2 = (j + 2) % NB
            if j < 2:
                if not first:
                    wait_scat(b2, ix_nxt, IB - 2 + j)
            else:
                wait_scat(b2, ix_cur, j - 2)
            if j == 1 and not (first or last):
                pltpu.async_copy(idx_hbm.at[wid, g + 1], ix_nxt, si)
            if j == IB - 2 and not last:
                pltpu.make_async_copy(idx_hbm.at[wid, g + 1], ix_nxt, si).wait()
            if j + 2 < IB:
                issue_gather(b2, ix_cur, j + 2)
            elif not last:
                issue_gather(b2, ix_nxt, j + 2 - IB)
        if last:
            wait_scat((IB - 2) % NB, ix_cur, IB - 2)
            wait_scat((IB - 1) % NB, ix_cur, IB - 1)

    @pl.loop(0, K)
    def _(r):
        for l in range(D // 16):
            zb[r, pl.ds(l * 16, 16)] = jnp.zeros((16,), jnp.float32)

    for t in range(RPS // K):
        pltpu.async_copy(zb, acc_sh.at[pl.ds(sid * RPS + t * K, K)], sz)
    pltpu.sync_copy(idx_hbm.at[wid, 0], ix0)
    pltpu.async_copy(idx_hbm.at[wid, 1], ix1, si)
    for t in range(RPS // K):
        pltpu.make_async_copy(zb, acc_sh.at[pl.ds(sid * RPS + t * K, K)], sz).wait()
    plsc.subcore_barrier()

    issue_gather(0, ix0, 0)
    issue_gather(1, ix0, 1)
    emit_group(ix0, ix1, 0, first=True)

    @pl.loop(0, (NG - 2) // 2)
    def _(i):
        g1 = 2 * i + 1
        emit_group(ix1, ix0, g1)
        emit_group(ix0, ix1, g1 + 1)

    emit_group(ix1, ix0, NG - 1, last=True)

    plsc.subcore_barrier()
    pltpu.sync_copy(acc_sh.at[pl.ds(sid * RPS, RPS)],
                    out_hbm.at[cid, pl.ds(sid * RPS, RPS)])


_agg = functools.partial(
    pl.kernel,
    out_type=jax.ShapeDtypeStruct((NC, NPAD, D), jnp.float32),
    mesh=_mesh,
    scratch_types=[
        pltpu.VMEM((2 * IB, K), jnp.int32),
        pltpu.VMEM((2 * IB, K), jnp.int32),
        pltpu.VMEM((K, D), jnp.float32),
        pltpu.VMEM((K, D), jnp.float32),
        pltpu.VMEM((K, D), jnp.float32),
        pltpu.VMEM((K, D), jnp.float32),
        pltpu.VMEM((K, D), jnp.float32),
        pltpu.VMEM_SHARED((NPAD, D), jnp.float32),
        pltpu.SemaphoreType.DMA,
        pltpu.SemaphoreType.DMA,
        pltpu.SemaphoreType.DMA,
        pltpu.SemaphoreType.DMA,
        pltpu.SemaphoreType.DMA,
        pltpu.SemaphoreType.DMA,
        pltpu.SemaphoreType.DMA,
        pltpu.SemaphoreType.DMA,
        pltpu.SemaphoreType.DMA,
        pltpu.SemaphoreType.DMA,
    ],
)(_agg_body)


def _mlp_body(p_ref, h_ref, w1_ref, b1_ref, w2_ref, b2_ref, o_ref):
    z = p_ref[0] + p_ref[1] + h_ref[...]
    y = jnp.maximum(
        jnp.dot(z, w1_ref[...], preferred_element_type=jnp.float32) + b1_ref[...], 0.0)
    o_ref[...] = jnp.maximum(
        jnp.dot(y, w2_ref[...], preferred_element_type=jnp.float32) + b2_ref[...], 0.0)


def _mlp_pool_body(p_ref, h_ref, w1_ref, b1_ref, w2_ref, b2_ref, bat_ref,
                   o_ref, pool_ref, acc_ref, cnt_ref):
    i = pl.program_id(0)
    z = p_ref[0] + p_ref[1] + h_ref[...]
    y = jnp.maximum(
        jnp.dot(z, w1_ref[...], preferred_element_type=jnp.float32) + b1_ref[...], 0.0)
    h2 = jnp.maximum(
        jnp.dot(y, w2_ref[...], preferred_element_type=jnp.float32) + b2_ref[...], 0.0)
    o_ref[...] = h2

    bn = h2.shape[0]
    ids = bat_ref[0]
    gids = lax.broadcasted_iota(jnp.int32, (G, bn), 0)
    oh = (gids == jnp.broadcast_to(ids, (G, bn))).astype(jnp.float32)
    part = jnp.dot(oh, h2, preferred_element_type=jnp.float32)
    csum = jnp.sum(oh, axis=1, keepdims=True)

    @pl.when(i == 0)
    def _():
        acc_ref[...] = part
        cnt_ref[...] = csum

    @pl.when(i > 0)
    def _():
        acc_ref[...] += part
        cnt_ref[...] += csum

    pool_ref[...] = acc_ref[...] / jnp.maximum(cnt_ref[...], 1.0)


_BN = 1000


def _mlp(p, h, w1t, b1, w2t, b2):
    return pl.pallas_call(
        _mlp_body,
        grid=(N // _BN,),
        in_specs=[
            pl.BlockSpec((NC, _BN, D), lambda i: (0, i, 0)),
            pl.BlockSpec((_BN, D), lambda i: (i, 0)),
            pl.BlockSpec((D, H), lambda i: (0, 0)),
            pl.BlockSpec((1, H), lambda i: (0, 0)),
            pl.BlockSpec((H, H), lambda i: (0, 0)),
            pl.BlockSpec((1, H), lambda i: (0, 0)),
        ],
        out_specs=pl.BlockSpec((_BN, D), lambda i: (i, 0)),
        out_shape=jax.ShapeDtypeStruct((N, D), jnp.float32),
    )(p, h, w1t, b1, w2t, b2)


def _mlp_pool(p, h, w1t, b1, w2t, b2, bat3):
    return pl.pallas_call(
        _mlp_pool_body,
        grid=(N // _BN,),
        in_specs=[
            pl.BlockSpec((NC, _BN, D), lambda i: (0, i, 0)),
            pl.BlockSpec((_BN, D), lambda i: (i, 0)),
            pl.BlockSpec((D, H), lambda i: (0, 0)),
            pl.BlockSpec((1, H), lambda i: (0, 0)),
            pl.BlockSpec((H, H), lambda i: (0, 0)),
            pl.BlockSpec((1, H), lambda i: (0, 0)),
            pl.BlockSpec((1, 1, _BN), lambda i: (i, 0, 0)),
        ],
        out_specs=[
            pl.BlockSpec((_BN, D), lambda i: (i, 0)),
            pl.BlockSpec((G, D), lambda i: (0, 0)),
        ],
        out_shape=[
            jax.ShapeDtypeStruct((N, D), jnp.float32),
            jax.ShapeDtypeStruct((G, D), jnp.float32),
        ],
        scratch_shapes=[
            pltpu.VMEM((G, D), jnp.float32),
            pltpu.VMEM((G, 1), jnp.float32),
        ],
    )(p, h, w1t, b1, w2t, b2, bat3)


def kernel(x, edge_index, batch, W1, b1, W2, b2, W3, b3, W4, b4):
    src = edge_index[0]
    dst = edge_index[1]
    padw = EPW - E // NW
    src2 = src.reshape(NW, E // NW)
    dst2 = dst.reshape(NW, E // NW)
    pad_src = jnp.zeros((NW, padw), jnp.int32)
    pad_dst = jnp.broadcast_to(
        N + (jnp.arange(padw, dtype=jnp.int32) % (NPAD - N)), (NW, padw))
    src4 = jnp.concatenate([src2, pad_src], axis=1).reshape(NW, NG, IB, K)
    dst4 = jnp.concatenate([dst2, pad_dst], axis=1).reshape(NW, NG, IB, K)
    idx4 = jnp.concatenate([src4, dst4], axis=2)
    bat3 = batch.reshape(N // _BN, 1, _BN)

    w1t, w2t, w3t, w4t = W1.T, W2.T, W3.T, W4.T
    b1r, b2r, b3r, b4r = (b.reshape(1, H) for b in (b1, b2, b3, b4))

    p1 = _agg(x, idx4)
    h1 = _mlp(p1, x, w1t, b1r, w2t, b2r)
    p2 = _agg(h1, idx4)
    h2, pool = _mlp_pool(p2, h1, w3t, b3r, w4t, b4r, bat3)
    return (pool, h2)


# speedup vs baseline: 3.8334x; 1.0504x over previous
import functools

import jax
import jax.numpy as jnp
from jax import lax
from jax.experimental import pallas as pl
from jax.experimental.pallas import tpu as pltpu
from jax.experimental.pallas import tpu_sc as plsc

N = 10000
E = 320000
D = 128
H = 128
G = 64

NC = 2
NS = 16
NW = NC * NS

K = 64
CHUNKS = 160
IB = 10
NG = CHUNKS // IB
NB = 5
LA = 3
EPW = CHUNKS * K
NPAD = 10240
RPS = NPAD // NS

_mesh = plsc.VectorSubcoreMesh(core_axis_name="c", subcore_axis_name="s")


def _agg_body(x_hbm, idx_hbm, out_hbm, ix0, ix1, b0, b1, b2, b3, b4,
              acc_sh, sg0, sg1, sg2, sg3, sg4, ss0, ss1, ss2, ss3, ss4,
              si, sz):
    zb = b0
    cid = lax.axis_index("c")
    sid = lax.axis_index("s")
    wid = cid * NS + sid

    bufs = (b0, b1, b2, b3, b4)
    gsem = (sg0, sg1, sg2, sg3, sg4)
    ssem = (ss0, ss1, ss2, ss3, ss4)

    def issue_gather(b, ix, row):
        pltpu.async_copy(x_hbm.at[ix.at[row]], bufs[b], gsem[b])

    def wait_gather(b, ix, row):
        pltpu.make_async_copy(x_hbm.at[ix.at[row]], bufs[b], gsem[b]).wait()

    def issue_scat(b, ix, row):
        pltpu.async_copy(bufs[b], acc_sh.at[ix.at[IB + row]], ssem[b], add=True)

    def wait_scat(b, ix, row):
        pltpu.make_async_copy(bufs[b], acc_sh.at[ix.at[IB + row]], ssem[b]).wait()

    def emit_group(ix_cur, ix_nxt, g, first=False, last=False):
        for j in range(IB):
            b = j % NB
            wait_gather(b, ix_cur, j)
            issue_scat(b, ix_cur, j)
            b2 = (j + LA) % NB
            if j < NB - LA:
                if not first:
                    wait_scat(b2, ix_nxt, IB - (NB - LA) + j)
            else:
                wait_scat(b2, ix_cur, j - (NB - LA))
            if j == 1 and not (first or last):
                pltpu.async_copy(idx_hbm.at[wid, g + 1], ix_nxt, si)
            if j == IB - LA and not last:
                pltpu.make_async_copy(idx_hbm.at[wid, g + 1], ix_nxt, si).wait()
            if j + LA < IB:
                issue_gather(b2, ix_cur, j + LA)
            elif not last:
                issue_gather(b2, ix_nxt, j + LA - IB)
        if last:
            for jj in range(IB - (NB - LA), IB):
                wait_scat(jj % NB, ix_cur, jj)

    @pl.loop(0, K)
    def _(r):
        for l in range(D // 16):
            zb[r, pl.ds(l * 16, 16)] = jnp.zeros((16,), jnp.float32)

    for t in range(RPS // K):
        pltpu.async_copy(zb, acc_sh.at[pl.ds(sid * RPS + t * K, K)], sz)
    pltpu.sync_copy(idx_hbm.at[wid, 0], ix0)
    pltpu.async_copy(idx_hbm.at[wid, 1], ix1, si)
    for t in range(RPS // K):
        pltpu.make_async_copy(zb, acc_sh.at[pl.ds(sid * RPS + t * K, K)], sz).wait()
    plsc.subcore_barrier()

    for b in range(LA):
        issue_gather(b, ix0, b)
    emit_group(ix0, ix1, 0, first=True)

    @pl.loop(0, (NG - 2) // 2)
    def _(i):
        g1 = 2 * i + 1
        emit_group(ix1, ix0, g1)
        emit_group(ix0, ix1, g1 + 1)

    emit_group(ix1, ix0, NG - 1, last=True)

    plsc.subcore_barrier()
    pltpu.sync_copy(acc_sh.at[pl.ds(sid * RPS, RPS)],
                    out_hbm.at[cid, pl.ds(sid * RPS, RPS)])


_agg = functools.partial(
    pl.kernel,
    out_type=jax.ShapeDtypeStruct((NC, NPAD, D), jnp.float32),
    mesh=_mesh,
    scratch_types=[
        pltpu.VMEM((2 * IB, K), jnp.int32),
        pltpu.VMEM((2 * IB, K), jnp.int32),
        pltpu.VMEM((K, D), jnp.float32),
        pltpu.VMEM((K, D), jnp.float32),
        pltpu.VMEM((K, D), jnp.float32),
        pltpu.VMEM((K, D), jnp.float32),
        pltpu.VMEM((K, D), jnp.float32),
        pltpu.VMEM_SHARED((NPAD, D), jnp.float32),
        pltpu.SemaphoreType.DMA,
        pltpu.SemaphoreType.DMA,
        pltpu.SemaphoreType.DMA,
        pltpu.SemaphoreType.DMA,
        pltpu.SemaphoreType.DMA,
        pltpu.SemaphoreType.DMA,
        pltpu.SemaphoreType.DMA,
        pltpu.SemaphoreType.DMA,
        pltpu.SemaphoreType.DMA,
        pltpu.SemaphoreType.DMA,
        pltpu.SemaphoreType.DMA,
        pltpu.SemaphoreType.DMA,
    ],
)(_agg_body)


def _mlp_body(p_ref, h_ref, w1_ref, b1_ref, w2_ref, b2_ref, o_ref):
    z = p_ref[0] + p_ref[1] + h_ref[...]
    y = jnp.maximum(
        jnp.dot(z, w1_ref[...], preferred_element_type=jnp.float32) + b1_ref[...], 0.0)
    o_ref[...] = jnp.maximum(
        jnp.dot(y, w2_ref[...], preferred_element_type=jnp.float32) + b2_ref[...], 0.0)


def _mlp_pool_body(p_ref, h_ref, w1_ref, b1_ref, w2_ref, b2_ref, bat_ref,
                   o_ref, pool_ref, acc_ref, cnt_ref):
    i = pl.program_id(0)
    z = p_ref[0] + p_ref[1] + h_ref[...]
    y = jnp.maximum(
        jnp.dot(z, w1_ref[...], preferred_element_type=jnp.float32) + b1_ref[...], 0.0)
    h2 = jnp.maximum(
        jnp.dot(y, w2_ref[...], preferred_element_type=jnp.float32) + b2_ref[...], 0.0)
    o_ref[...] = h2

    bn = h2.shape[0]
    ids = bat_ref[0]
    gids = lax.broadcasted_iota(jnp.int32, (G, bn), 0)
    oh = (gids == jnp.broadcast_to(ids, (G, bn))).astype(jnp.float32)
    part = jnp.dot(oh, h2, preferred_element_type=jnp.float32)
    csum = jnp.sum(oh, axis=1, keepdims=True)

    @pl.when(i == 0)
    def _():
        acc_ref[...] = part
        cnt_ref[...] = csum

    @pl.when(i > 0)
    def _():
        acc_ref[...] += part
        cnt_ref[...] += csum

    pool_ref[...] = acc_ref[...] / jnp.maximum(cnt_ref[...], 1.0)


_BN = 1000


def _mlp(p, h, w1t, b1, w2t, b2):
    return pl.pallas_call(
        _mlp_body,
        grid=(N // _BN,),
        in_specs=[
            pl.BlockSpec((NC, _BN, D), lambda i: (0, i, 0)),
            pl.BlockSpec((_BN, D), lambda i: (i, 0)),
            pl.BlockSpec((D, H), lambda i: (0, 0)),
            pl.BlockSpec((1, H), lambda i: (0, 0)),
            pl.BlockSpec((H, H), lambda i: (0, 0)),
            pl.BlockSpec((1, H), lambda i: (0, 0)),
        ],
        out_specs=pl.BlockSpec((_BN, D), lambda i: (i, 0)),
        out_shape=jax.ShapeDtypeStruct((N, D), jnp.float32),
    )(p, h, w1t, b1, w2t, b2)


def _mlp_pool(p, h, w1t, b1, w2t, b2, bat3):
    return pl.pallas_call(
        _mlp_pool_body,
        grid=(N // _BN,),
        in_specs=[
            pl.BlockSpec((NC, _BN, D), lambda i: (0, i, 0)),
            pl.BlockSpec((_BN, D), lambda i: (i, 0)),
            pl.BlockSpec((D, H), lambda i: (0, 0)),
            pl.BlockSpec((1, H), lambda i: (0, 0)),
            pl.BlockSpec((H, H), lambda i: (0, 0)),
            pl.BlockSpec((1, H), lambda i: (0, 0)),
            pl.BlockSpec((1, 1, _BN), lambda i: (i, 0, 0)),
        ],
        out_specs=[
            pl.BlockSpec((_BN, D), lambda i: (i, 0)),
            pl.BlockSpec((G, D), lambda i: (0, 0)),
        ],
        out_shape=[
            jax.ShapeDtypeStruct((N, D), jnp.float32),
            jax.ShapeDtypeStruct((G, D), jnp.float32),
        ],
        scratch_shapes=[
            pltpu.VMEM((G, D), jnp.float32),
            pltpu.VMEM((G, 1), jnp.float32),
        ],
    )(p, h, w1t, b1, w2t, b2, bat3)


def kernel(x, edge_index, batch, W1, b1, W2, b2, W3, b3, W4, b4):
    src = edge_index[0]
    dst = edge_index[1]
    padw = EPW - E // NW
    src2 = src.reshape(NW, E // NW)
    dst2 = dst.reshape(NW, E // NW)
    pad_src = jnp.zeros((NW, padw), jnp.int32)
    pad_dst = jnp.broadcast_to(
        N + (jnp.arange(padw, dtype=jnp.int32) % (NPAD - N)), (NW, padw))
    src4 = jnp.concatenate([src2, pad_src], axis=1).reshape(NW, NG, IB, K)
    dst4 = jnp.concatenate([dst2, pad_dst], axis=1).reshape(NW, NG, IB, K)
    idx4 = jnp.concatenate([src4, dst4], axis=2)
    bat3 = batch.reshape(N // _BN, 1, _BN)

    w1t, w2t, w3t, w4t = W1.T, W2.T, W3.T, W4.T
    b1r, b2r, b3r, b4r = (b.reshape(1, H) for b in (b1, b2, b3, b4))

    p1 = _agg(x, idx4)
    h1 = _mlp(p1, x, w1t, b1r, w2t, b2r)
    p2 = _agg(h1, idx4)
    h2, pool = _mlp_pool(p2, h1, w3t, b3r, w4t, b4r, bat3)
    return (pool, h2)


# speedup vs baseline: 3.8628x; 1.0077x over previous
import functools

import jax
import jax.numpy as jnp
from jax import lax
from jax.experimental import pallas as pl
from jax.experimental.pallas import tpu as pltpu
from jax.experimental.pallas import tpu_sc as plsc

N = 10000
E = 320000
D = 128
H = 128
G = 64

NC = 2
NS = 16
NW = NC * NS

K = 32
CHUNKS = 320
IB = 16
NG = CHUNKS // IB
NB = 8
LA = 6
EPW = CHUNKS * K
NPAD = 10240
RPS = NPAD // NS

_mesh = plsc.VectorSubcoreMesh(core_axis_name="c", subcore_axis_name="s")


def _agg_body(x_hbm, idx_hbm, out_hbm, ix0, ix1, b0, b1, b2, b3, b4, b5,
              b6, b7, acc_sh, sg0, sg1, sg2, sg3, sg4, sg5, sg6, sg7,
              ss0, ss1, ss2, ss3, ss4, ss5, ss6, ss7, si, sz):
    zb = b0
    cid = lax.axis_index("c")
    sid = lax.axis_index("s")
    wid = cid * NS + sid

    bufs = (b0, b1, b2, b3, b4, b5, b6, b7)
    gsem = (sg0, sg1, sg2, sg3, sg4, sg5, sg6, sg7)
    ssem = (ss0, ss1, ss2, ss3, ss4, ss5, ss6, ss7)

    def issue_gather(b, ix, row):
        pltpu.async_copy(x_hbm.at[ix.at[row]], bufs[b], gsem[b])

    def wait_gather(b, ix, row):
        pltpu.make_async_copy(x_hbm.at[ix.at[row]], bufs[b], gsem[b]).wait()

    def issue_scat(b, ix, row):
        pltpu.async_copy(bufs[b], acc_sh.at[ix.at[IB + row]], ssem[b], add=True)

    def wait_scat(b, ix, row):
        pltpu.make_async_copy(bufs[b], acc_sh.at[ix.at[IB + row]], ssem[b]).wait()

    def emit_group(ix_cur, ix_nxt, g, first=False, last=False):
        for j in range(IB):
            b = j % NB
            wait_gather(b, ix_cur, j)
            issue_scat(b, ix_cur, j)
            b2 = (j + LA) % NB
            if j < NB - LA:
                if not first:
                    wait_scat(b2, ix_nxt, IB - (NB - LA) + j)
            else:
                wait_scat(b2, ix_cur, j - (NB - LA))
            if j == 1 and not (first or last):
                pltpu.async_copy(idx_hbm.at[wid, g + 1], ix_nxt, si)
            if j == IB - LA and not last:
                pltpu.make_async_copy(idx_hbm.at[wid, g + 1], ix_nxt, si).wait()
            if j + LA < IB:
                issue_gather(b2, ix_cur, j + LA)
            elif not last:
                issue_gather(b2, ix_nxt, j + LA - IB)
        if last:
            for jj in range(IB - (NB - LA), IB):
                wait_scat(jj % NB, ix_cur, jj)

    @pl.loop(0, K)
    def _(r):
        for l in range(D // 16):
            zb[r, pl.ds(l * 16, 16)] = jnp.zeros((16,), jnp.float32)

    for t in range(RPS // K):
        pltpu.async_copy(zb, acc_sh.at[pl.ds(sid * RPS + t * K, K)], sz)
    pltpu.sync_copy(idx_hbm.at[wid, 0], ix0)
    pltpu.async_copy(idx_hbm.at[wid, 1], ix1, si)
    for t in range(RPS // K):
        pltpu.make_async_copy(zb, acc_sh.at[pl.ds(sid * RPS + t * K, K)], sz).wait()
    plsc.subcore_barrier()

    for b in range(LA):
        issue_gather(b, ix0, b)
    emit_group(ix0, ix1, 0, first=True)

    @pl.loop(0, (NG - 2) // 2)
    def _(i):
        g1 = 2 * i + 1
        emit_group(ix1, ix0, g1)
        emit_group(ix0, ix1, g1 + 1)

    emit_group(ix1, ix0, NG - 1, last=True)

    plsc.subcore_barrier()
    pltpu.sync_copy(acc_sh.at[pl.ds(sid * RPS, RPS)],
                    out_hbm.at[cid, pl.ds(sid * RPS, RPS)])


_agg = functools.partial(
    pl.kernel,
    out_type=jax.ShapeDtypeStruct((NC, NPAD, D), jnp.float32),
    mesh=_mesh,
    scratch_types=[
        pltpu.VMEM((2 * IB, K), jnp.int32),
        pltpu.VMEM((2 * IB, K), jnp.int32),
        pltpu.VMEM((K, D), jnp.float32),
        pltpu.VMEM((K, D), jnp.float32),
        pltpu.VMEM((K, D), jnp.float32),
        pltpu.VMEM((K, D), jnp.float32),
        pltpu.VMEM((K, D), jnp.float32),
        pltpu.VMEM((K, D), jnp.float32),
        pltpu.VMEM((K, D), jnp.float32),
        pltpu.VMEM((K, D), jnp.float32),
        pltpu.VMEM_SHARED((NPAD, D), jnp.float32),
    ] + [pltpu.SemaphoreType.DMA] * 18,
)(_agg_body)


def _mlp_body(p_ref, h_ref, w1_ref, b1_ref, w2_ref, b2_ref, o_ref):
    z = p_ref[0] + p_ref[1] + h_ref[...]
    y = jnp.maximum(
        jnp.dot(z, w1_ref[...], preferred_element_type=jnp.float32) + b1_ref[...], 0.0)
    o_ref[...] = jnp.maximum(
        jnp.dot(y, w2_ref[...], preferred_element_type=jnp.float32) + b2_ref[...], 0.0)


def _mlp_pool_body(p_ref, h_ref, w1_ref, b1_ref, w2_ref, b2_ref, bat_ref,
                   o_ref, pool_ref, acc_ref, cnt_ref):
    i = pl.program_id(0)
    z = p_ref[0] + p_ref[1] + h_ref[...]
    y = jnp.maximum(
        jnp.dot(z, w1_ref[...], preferred_element_type=jnp.float32) + b1_ref[...], 0.0)
    h2 = jnp.maximum(
        jnp.dot(y, w2_ref[...], preferred_element_type=jnp.float32) + b2_ref[...], 0.0)
    o_ref[...] = h2

    bn = h2.shape[0]
    ids = bat_ref[0]
    gids = lax.broadcasted_iota(jnp.int32, (G, bn), 0)
    oh = (gids == jnp.broadcast_to(ids, (G, bn))).astype(jnp.float32)
    part = jnp.dot(oh, h2, preferred_element_type=jnp.float32)
    csum = jnp.sum(oh, axis=1, keepdims=True)

    @pl.when(i == 0)
    def _():
        acc_ref[...] = part
        cnt_ref[...] = csum

    @pl.when(i > 0)
    def _():
        acc_ref[...] += part
        cnt_ref[...] += csum

    pool_ref[...] = acc_ref[...] / jnp.maximum(cnt_ref[...], 1.0)


_BN = 1000


def _mlp(p, h, w1t, b1, w2t, b2):
    return pl.pallas_call(
        _mlp_body,
        grid=(N // _BN,),
        in_specs=[
            pl.BlockSpec((NC, _BN, D), lambda i: (0, i, 0)),
            pl.BlockSpec((_BN, D), lambda i: (i, 0)),
            pl.BlockSpec((D, H), lambda i: (0, 0)),
            pl.BlockSpec((1, H), lambda i: (0, 0)),
            pl.BlockSpec((H, H), lambda i: (0, 0)),
            pl.BlockSpec((1, H), lambda i: (0, 0)),
        ],
        out_specs=pl.BlockSpec((_BN, D), lambda i: (i, 0)),
        out_shape=jax.ShapeDtypeStruct((N, D), jnp.float32),
    )(p, h, w1t, b1, w2t, b2)


def _mlp_pool(p, h, w1t, b1, w2t, b2, bat3):
    return pl.pallas_call(
        _mlp_pool_body,
        grid=(N // _BN,),
        in_specs=[
            pl.BlockSpec((NC, _BN, D), lambda i: (0, i, 0)),
            pl.BlockSpec((_BN, D), lambda i: (i, 0)),
            pl.BlockSpec((D, H), lambda i: (0, 0)),
            pl.BlockSpec((1, H), lambda i: (0, 0)),
            pl.BlockSpec((H, H), lambda i: (0, 0)),
            pl.BlockSpec((1, H), lambda i: (0, 0)),
            pl.BlockSpec((1, 1, _BN), lambda i: (i, 0, 0)),
        ],
        out_specs=[
            pl.BlockSpec((_BN, D), lambda i: (i, 0)),
            pl.BlockSpec((G, D), lambda i: (0, 0)),
        ],
        out_shape=[
            jax.ShapeDtypeStruct((N, D), jnp.float32),
            jax.ShapeDtypeStruct((G, D), jnp.float32),
        ],
        scratch_shapes=[
            pltpu.VMEM((G, D), jnp.float32),
            pltpu.VMEM((G, 1), jnp.float32),
        ],
    )(p, h, w1t, b1, w2t, b2, bat3)


def kernel(x, edge_index, batch, W1, b1, W2, b2, W3, b3, W4, b4):
    src = edge_index[0]
    dst = edge_index[1]
    padw = EPW - E // NW
    src2 = src.reshape(NW, E // NW)
    dst2 = dst.reshape(NW, E // NW)
    pad_src = jnp.zeros((NW, padw), jnp.int32)
    pad_dst = jnp.broadcast_to(
        N + (jnp.arange(padw, dtype=jnp.int32) % (NPAD - N)), (NW, padw))
    src4 = jnp.concatenate([src2, pad_src], axis=1).reshape(NW, NG, IB, K)
    dst4 = jnp.concatenate([dst2, pad_dst], axis=1).reshape(NW, NG, IB, K)
    idx4 = jnp.concatenate([src4, dst4], axis=2)
    bat3 = batch.reshape(N // _BN, 1, _BN)

    w1t, w2t, w3t, w4t = W1.T, W2.T, W3.T, W4.T
    b1r, b2r, b3r, b4r = (b.reshape(1, H) for b in (b1, b2, b3, b4))

    p1 = _agg(x, idx4)
    h1 = _mlp(p1, x, w1t, b1r, w2t, b2r)
    p2 = _agg(h1, idx4)
    h2, pool = _mlp_pool(p2, h1, w3t, b3r, w4t, b4r, bat3)
    return (pool, h2)
